# baseline (speedup 1.0000x reference)
import jax
import jax.numpy as jnp
from jax.experimental import pallas as pl

D_MODEL = 2048
D_SPACE = 64
GROUPS5 = (
    (0, 0, 256, 8),
    (256, 256, 256, 8),
    (512, 512, 256, 8),
    (768, 768, 256, 8),
    (1024, 1024, 256, 8),
)
GROUPS2 = (
    (0, 1280, 1024, 4),
    (1024, 2304, 1024, 4),
)
N_OUT = 3328
N_PROJ = 512
TILE = 256


def _epilogue(l, k):
    m = jnp.max(l, axis=-1, keepdims=True)
    cur, mv = l, m
    for _ in range(k - 1):
        cur = jnp.where(cur >= mv, -jnp.inf, cur)
        mv = jnp.max(cur, axis=-1, keepdims=True)
    ex = jnp.exp(l - m)
    mex = jnp.where(l >= mv, ex, 0.0)
    se = jnp.sum(mex, axis=-1, keepdims=True)
    return mex * (1.0 / se)


def _router_kernel(x_ref, w_ref, b_ref, e5_ref, e2_ref, o_ref):
    xb = x_ref[...].astype(jnp.bfloat16)
    h = jax.lax.dot_general(xb, w_ref[...], (((1,), (1,)), ((), ())),
                            preferred_element_type=jnp.float32)
    h = (h + b_ref[...]).astype(jnp.bfloat16)
    logits5 = jnp.dot(h[:, 0:384], e5_ref[...],
                      preferred_element_type=jnp.float32)
    logits2 = jnp.dot(h[:, 384:512], e2_ref[...],
                      preferred_element_type=jnp.float32)

    for src, out_off, width, k in GROUPS5:
        o_ref[:, out_off:out_off + width] = _epilogue(
            logits5[:, src:src + width], k)
    for src, out_off, width, k in GROUPS2:
        o_ref[:, out_off:out_off + width] = _epilogue(
            logits2[:, src:src + width], k)


@jax.jit
def kernel(x, importance, proj_all_W, proj_all_b, proj_fk_W, proj_fk_b,
           proj_rk_W, proj_rk_b, neuron_emb):
    del importance
    b, s, d = x.shape
    n_tok = b * s
    xf = x.reshape(n_tok, d)

    w_cat = jnp.zeros((N_PROJ, d), dtype=jnp.bfloat16)
    w_cat = w_cat.at[0:320].set(proj_all_W.astype(jnp.bfloat16))
    w_cat = w_cat.at[384:448].set(proj_fk_W.astype(jnp.bfloat16))
    w_cat = w_cat.at[448:512].set(proj_rk_W.astype(jnp.bfloat16))
    b_cat = jnp.zeros((1, N_PROJ), dtype=jnp.float32)
    b_cat = b_cat.at[0, 0:320].set(proj_all_b)
    b_cat = b_cat.at[0, 384:448].set(proj_fk_b)
    b_cat = b_cat.at[0, 448:512].set(proj_rk_b)

    norm = jnp.maximum(jnp.linalg.norm(neuron_emb, axis=-1, keepdims=True),
                       1e-12)
    emb_norm = (neuron_emb / norm).astype(jnp.bfloat16)

    e5 = jnp.zeros((384, 1280), dtype=jnp.bfloat16)
    chunks5 = (emb_norm[0:256], emb_norm[256:512], emb_norm[512:768],
               emb_norm[512:768], emb_norm[768:1024])
    for g in range(5):
        e5 = e5.at[64 * g:64 * (g + 1), 256 * g:256 * (g + 1)].set(
            chunks5[g].T)
    e2 = jnp.zeros((128, 2048), dtype=jnp.bfloat16)
    e2 = e2.at[0:64, 0:1024].set(emb_norm[1024:2048].T)
    e2 = e2.at[64:128, 1024:2048].set(emb_norm[2048:3072].T)

    grid = (n_tok // TILE,)
    out = pl.pallas_call(
        _router_kernel,
        grid=grid,
        in_specs=[
            pl.BlockSpec((TILE, d), lambda i: (i, 0)),
            pl.BlockSpec((N_PROJ, d), lambda i: (0, 0)),
            pl.BlockSpec((1, N_PROJ), lambda i: (0, 0)),
            pl.BlockSpec((384, 1280), lambda i: (0, 0)),
            pl.BlockSpec((128, 2048), lambda i: (0, 0)),
        ],
        out_specs=pl.BlockSpec((TILE, N_OUT), lambda i: (i, 0)),
        out_shape=jax.ShapeDtypeStruct((n_tok, N_OUT), jnp.float32),
    )(xf, w_cat, b_cat, e5, e2)
    return out.reshape(b, s, N_OUT)


# speedup vs baseline: 1.0372x; 1.0372x over previous
import jax
import jax.numpy as jnp
from jax.experimental import pallas as pl

D_MODEL = 2048
D_SPACE = 64
GROUPS = (
    (0, 256, 8),
    (256, 256, 8),
    (512, 256, 8),
    (768, 256, 8),
    (1024, 256, 8),
    (1280, 1024, 4),
    (2304, 1024, 4),
)
N_OUT = 3328
N_PROJ = 448
TILE = 512


def _router_kernel(x_ref, w_ref, b_ref, e_ref, o_ref):
    xb = x_ref[...].astype(jnp.bfloat16)
    h = jnp.dot(xb, w_ref[...], preferred_element_type=jnp.float32)
    h = (h + b_ref[...]).astype(jnp.bfloat16)
    logits = jnp.dot(h, e_ref[...], preferred_element_type=jnp.float32)

    for off, width, k in GROUPS:
        l = logits[:, off:off + width]
        m = jnp.max(l, axis=-1, keepdims=True)
        cur, mv = l, m
        for _ in range(k - 1):
            cur = jnp.where(cur >= mv, -jnp.inf, cur)
            mv = jnp.max(cur, axis=-1, keepdims=True)
        ex = jnp.exp(l - m)
        mex = jnp.where(l >= mv, ex, 0.0)
        se = jnp.sum(mex, axis=-1, keepdims=True)
        o_ref[:, off:off + width] = mex * (1.0 / se)


@jax.jit
def kernel(x, importance, proj_all_W, proj_all_b, proj_fk_W, proj_fk_b,
           proj_rk_W, proj_rk_b, neuron_emb):
    del importance
    b, s, d = x.shape
    n_tok = b * s
    xf = x.reshape(n_tok, d)

    w_cat = jnp.concatenate([proj_all_W, proj_fk_W, proj_rk_W],
                            axis=0).T.astype(jnp.bfloat16)
    b_cat = jnp.concatenate([proj_all_b, proj_fk_b, proj_rk_b],
                            axis=0).reshape(1, N_PROJ)

    norm = jnp.maximum(jnp.linalg.norm(neuron_emb, axis=-1, keepdims=True),
                       1e-12)
    emb_norm = neuron_emb / norm

    emb_chunks = (
        emb_norm[0:256], emb_norm[256:512], emb_norm[512:768],
        emb_norm[512:768], emb_norm[768:1024], emb_norm[1024:2048],
        emb_norm[2048:3072],
    )
    e_bd = jnp.zeros((N_PROJ, N_OUT), dtype=jnp.float32)
    for g, (off, width, _) in enumerate(GROUPS):
        e_bd = e_bd.at[64 * g:64 * (g + 1), off:off + width].set(
            emb_chunks[g].T)
    e_bd = e_bd.astype(jnp.bfloat16)

    grid = (n_tok // TILE,)
    out = pl.pallas_call(
        _router_kernel,
        grid=grid,
        in_specs=[
            pl.BlockSpec((TILE, d), lambda i: (i, 0)),
            pl.BlockSpec((d, N_PROJ), lambda i: (0, 0)),
            pl.BlockSpec((1, N_PROJ), lambda i: (0, 0)),
            pl.BlockSpec((N_PROJ, N_OUT), lambda i: (0, 0)),
        ],
        out_specs=pl.BlockSpec((TILE, N_OUT), lambda i: (i, 0)),
        out_shape=jax.ShapeDtypeStruct((n_tok, N_OUT), jnp.float32),
    )(xf, w_cat, b_cat, e_bd)
    return out.reshape(b, s, N_OUT)


# speedup vs baseline: 1.0769x; 1.0382x over previous
import jax
import jax.numpy as jnp
from jax.experimental import pallas as pl
from jax.experimental.pallas import tpu as pltpu

D_MODEL = 2048
D_SPACE = 64
GROUPS = (
    (0, 256, 8),
    (256, 256, 8),
    (512, 256, 8),
    (768, 256, 8),
    (1024, 256, 8),
    (1280, 1024, 4),
    (2304, 1024, 4),
)
EMB_ROWS = ((0, 256), (256, 512), (512, 768), (512, 768), (768, 1024),
            (1024, 2048), (2048, 3072))
N_OUT = 3328
N_PROJ = 448
TILE = 256


def _router_kernel(x_ref, w_ref, b_ref, emb_ref, o_ref, w_s, e_s):
    @pl.when(pl.program_id(0) == 0)
    def _prep():
        w_s[...] = w_ref[...].T.astype(jnp.bfloat16)
        emb = emb_ref[...]
        norm = jnp.maximum(
            jnp.sqrt(jnp.sum(emb * emb, axis=-1, keepdims=True)), 1e-12)
        emb_n = (emb / norm).astype(jnp.bfloat16)
        e_s[...] = jnp.zeros((N_PROJ, N_OUT), dtype=jnp.bfloat16)
        for g, (off, width, _) in enumerate(GROUPS):
            a, bb = EMB_ROWS[g]
            e_s[64 * g:64 * (g + 1), off:off + width] = emb_n[a:bb].T

    xb = x_ref[...].astype(jnp.bfloat16)
    h = jnp.dot(xb, w_s[...], preferred_element_type=jnp.float32)
    h = (h + b_ref[...]).astype(jnp.bfloat16)
    logits = jnp.dot(h, e_s[...], preferred_element_type=jnp.float32)

    for off, width, k in GROUPS:
        l = logits[:, off:off + width]
        m = jnp.max(l, axis=-1, keepdims=True)
        cur, mv = l, m
        for _ in range(k - 1):
            cur = jnp.where(cur >= mv, -jnp.inf, cur)
            mv = jnp.max(cur, axis=-1, keepdims=True)
        ex = jnp.exp(l - m)
        mex = jnp.where(l >= mv, ex, 0.0)
        se = jnp.sum(mex, axis=-1, keepdims=True)
        o_ref[:, off:off + width] = mex * (1.0 / se)


@jax.jit
def kernel(x, importance, proj_all_W, proj_all_b, proj_fk_W, proj_fk_b,
           proj_rk_W, proj_rk_b, neuron_emb):
    del importance
    b, s, d = x.shape
    n_tok = b * s
    xf = x.reshape(n_tok, d)

    w_cat = jnp.concatenate([proj_all_W, proj_fk_W, proj_rk_W], axis=0)
    b_cat = jnp.concatenate([proj_all_b, proj_fk_b, proj_rk_b],
                            axis=0).reshape(1, N_PROJ)

    grid = (n_tok // TILE,)
    out = pl.pallas_call(
        _router_kernel,
        grid=grid,
        in_specs=[
            pl.BlockSpec((TILE, d), lambda i: (i, 0)),
            pl.BlockSpec((N_PROJ, d), lambda i: (0, 0)),
            pl.BlockSpec((1, N_PROJ), lambda i: (0, 0)),
            pl.BlockSpec((3072, D_SPACE), lambda i: (0, 0)),
        ],
        out_specs=pl.BlockSpec((TILE, N_OUT), lambda i: (i, 0)),
        out_shape=jax.ShapeDtypeStruct((n_tok, N_OUT), jnp.float32),
        scratch_shapes=[
            pltpu.VMEM((d, N_PROJ), jnp.bfloat16),
            pltpu.VMEM((N_PROJ, N_OUT), jnp.bfloat16),
        ],
    )(xf, w_cat, b_cat, neuron_emb)
    return out.reshape(b, s, N_OUT)
